# baseline (speedup 1.0000x reference)
import functools

import jax
import jax.numpy as jnp
from jax import lax
from jax.experimental import pallas as pl
from jax.experimental.pallas import tpu as pltpu
from jax.experimental.pallas import tpu_sc as plsc

_INFO = plsc.get_sparse_core_info()
_NC = _INFO.num_cores
_NS = _INFO.num_subcores
_NW = _NC * _NS
_NBUF = 3
_CHUNK = 40


def _chunk_schedule(rows):
    sched = []
    off = 0
    while off < rows:
        size = min(_CHUNK, rows - off)
        sched.append((off, size))
        off += size
    return sched


def _gather_call(idx, table, rows_per_w):
    n_tokens = _NW * rows_per_w
    d = table.shape[1]
    sched = _chunk_schedule(rows_per_w)
    n_steps = len(sched)
    mesh = plsc.VectorSubcoreMesh(core_axis_name="c", subcore_axis_name="s")

    @functools.partial(
        pl.kernel,
        mesh=mesh,
        out_type=jax.ShapeDtypeStruct((n_tokens, d), table.dtype),
        scratch_types=[
            pltpu.VMEM((rows_per_w,), jnp.int32),
            pltpu.VMEM((_NBUF, _CHUNK, d), table.dtype),
        ]
        + [pltpu.SemaphoreType.DMA] * (2 * _NBUF),
    )
    def run(idx_hbm, table_hbm, out_hbm, idx_v, bufs, *sems):
        wid = lax.axis_index("s") * _NC + lax.axis_index("c")
        base = wid * rows_per_w
        gsems = sems[:_NBUF]
        ssems = sems[_NBUF:]

        pltpu.sync_copy(idx_hbm.at[wid], idx_v)

        def gather_copy(c):
            b = c % _NBUF
            off, size = sched[c]
            return pltpu.make_async_copy(
                table_hbm.at[idx_v.at[pl.ds(off, size)]],
                bufs.at[b, pl.ds(0, size)],
                gsems[b],
            )

        def out_copy(c):
            b = c % _NBUF
            off, size = sched[c]
            return pltpu.make_async_copy(
                bufs.at[b, pl.ds(0, size)],
                out_hbm.at[pl.ds(base + off, size)],
                ssems[b],
            )

        for c in range(min(_NBUF - 1, n_steps)):
            gather_copy(c).start()
        for c in range(n_steps):
            if c + _NBUF - 1 < n_steps:
                if c - 1 >= 0:
                    out_copy(c - 1).wait()
                gather_copy(c + _NBUF - 1).start()
            gather_copy(c).wait()
            out_copy(c).start()
        for c in range(max(0, n_steps - _NBUF), n_steps):
            out_copy(c).wait()

    return run(idx, table)


def kernel(tokens, W_E):
    b, s = tokens.shape
    v, d = W_E.shape
    n = b * s
    rows_per_w = n // _NW
    idx = tokens.reshape(_NW, rows_per_w)
    out = _gather_call(idx, W_E, rows_per_w)
    return out.reshape(b, s, d)


# speedup vs baseline: 1.0053x; 1.0053x over previous
import functools

import jax
import jax.numpy as jnp
from jax import lax
from jax.experimental import pallas as pl
from jax.experimental.pallas import tpu as pltpu
from jax.experimental.pallas import tpu_sc as plsc

_NBUF = 3
_CHUNK = 40


def _chunk_schedule(rows):
    sched = []
    off = 0
    while off < rows:
        size = min(_CHUNK, rows - off)
        sched.append((off, size))
        off += size
    return sched


def _gather_call(idx, table, rows_per_w, nc, ns):
    n_tokens = nc * ns * rows_per_w
    d = table.shape[1]
    sched = _chunk_schedule(rows_per_w)
    n_steps = len(sched)
    mesh = plsc.VectorSubcoreMesh(core_axis_name="c", subcore_axis_name="s")

    @functools.partial(
        pl.kernel,
        mesh=mesh,
        out_type=jax.ShapeDtypeStruct((n_tokens, d), table.dtype),
        scratch_types=[
            pltpu.VMEM((rows_per_w,), jnp.int32),
            pltpu.VMEM((_NBUF, _CHUNK, d), table.dtype),
        ]
        + [pltpu.SemaphoreType.DMA] * (2 * _NBUF),
    )
    def run(idx_hbm, table_hbm, out_hbm, idx_v, bufs, *sems):
        wid = lax.axis_index("s") * nc + lax.axis_index("c")
        base = wid * rows_per_w
        gsems = sems[:_NBUF]
        ssems = sems[_NBUF:]

        pltpu.sync_copy(idx_hbm.at[wid], idx_v)

        def gather_copy(c):
            b = c % _NBUF
            off, size = sched[c]
            return pltpu.make_async_copy(
                table_hbm.at[idx_v.at[pl.ds(off, size)]],
                bufs.at[b, pl.ds(0, size)],
                gsems[b],
            )

        def out_copy(c):
            b = c % _NBUF
            off, size = sched[c]
            return pltpu.make_async_copy(
                bufs.at[b, pl.ds(0, size)],
                out_hbm.at[pl.ds(base + off, size)],
                ssems[b],
            )

        for c in range(min(_NBUF - 1, n_steps)):
            gather_copy(c).start()
        for c in range(n_steps):
            if c + _NBUF - 1 < n_steps:
                if c - 1 >= 0:
                    out_copy(c - 1).wait()
                gather_copy(c + _NBUF - 1).start()
            gather_copy(c).wait()
            out_copy(c).start()
        for c in range(max(0, n_steps - _NBUF), n_steps):
            out_copy(c).wait()

    return run(idx, table)


def kernel(tokens, W_E):
    b, s = tokens.shape
    v, d = W_E.shape
    n = b * s
    info = plsc.get_sparse_core_info()
    nc, ns = info.num_cores, info.num_subcores
    nw = nc * ns
    rows_per_w = n // nw
    idx = tokens.reshape(nw, rows_per_w)
    out = _gather_call(idx, W_E, rows_per_w, nc, ns)
    return out.reshape(b, s, d)
